# baseline (speedup 1.0000x reference)
import functools

import jax
import jax.numpy as jnp
from jax import lax
from jax.experimental import pallas as pl
from jax.experimental.pallas import tpu as pltpu
from jax.experimental.pallas import tpu_sc as plsc

F = 256
FW = 512
NC, NS, L = 2, 16, 16
NT = NC * NS
PH = 4
OWN = 80
B = 16
FB = 384
CPB = FB // B
SBL = 784
SC_CHUNK = 2048
RB = 10


def _dense_body(node_ref, rel_ref, wt_ref, wq_ref, lw_ref, elw_ref,
                asrc_ref, adst_ref, arel_ref, csum_ref, l_ref, el_ref):
    wq = wq_ref[...]
    nb = node_ref[...]
    w1 = wt_ref[0:F, :]
    w2 = wt_ref[F:2 * F, :]
    w3 = wt_ref[2 * F:3 * F, :]
    p1 = jnp.dot(nb, w1, preferred_element_type=jnp.float32)
    asrc_ref[:, 0:F] = p1
    asrc_ref[:, F:FW] = jnp.dot(p1, wq, preferred_element_type=jnp.float32)
    p3 = jnp.dot(nb, w3, preferred_element_type=jnp.float32)
    adst_ref[:, 0:F] = p3
    adst_ref[:, F:FW] = jnp.dot(p3, wq, preferred_element_type=jnp.float32)
    pr = jnp.dot(rel_ref[...], w2, preferred_element_type=jnp.float32)
    arel_ref[:, 0:F] = pr
    arel_ref[:, F:FW] = jnp.dot(pr, wq, preferred_element_type=jnp.float32)
    csum_ref[...] = jnp.sum(wq, axis=0, keepdims=True)
    l_ref[...] = jnp.dot(nb, lw_ref[...], preferred_element_type=jnp.float32)
    el_ref[...] = jnp.dot(nb, elw_ref[...], preferred_element_type=jnp.float32)


def _final_body(nd_ref, norm_ref, l_ref, el_ref, h_ref):
    nd = nd_ref[...]
    num = nd[:, 0:F]
    den = nd[:, F:FW]
    agg = jnp.where(den > 0, num / jnp.maximum(den, 1e-30), 0.0)
    loop = jnp.where(den[:, 0:1] > 0, l_ref[...], el_ref[...])
    h_ref[...] = agg * norm_ref[...] + loop


def _make_edge_kernel(n_scan, cap):
    mesh = plsc.VectorSubcoreMesh(
        core_axis_name="c", subcore_axis_name="s",
        num_cores=NC, num_subcores=NS)

    @functools.partial(
        pl.kernel,
        out_type=(
            jax.ShapeDtypeStruct((NT * PH * OWN, FW), jnp.float32),
            jax.ShapeDtypeStruct((NT * PH, cap), jnp.int32),
        ),
        mesh=mesh,
        compiler_params=pltpu.CompilerParams(needs_layout_passes=False),
        scratch_types=[
            pltpu.VMEM((SC_CHUNK,), jnp.int32),
            pltpu.VMEM((SC_CHUNK,), jnp.int32),
            pltpu.VMEM((SBL,), jnp.int32),
            pltpu.VMEM((FB,), jnp.int32),
            pltpu.VMEM((B, 128), jnp.int32),
            pltpu.VMEM((B, 128), jnp.int32),
            pltpu.VMEM((B,), jnp.int32),
            pltpu.VMEM((B,), jnp.int32),
            pltpu.VMEM((B,), jnp.int32),
            pltpu.VMEM((B,), jnp.int32),
            pltpu.VMEM((B,), jnp.int32),
            pltpu.VMEM((B,), jnp.int32),
            pltpu.VMEM((B + L,), jnp.int32),
            pltpu.VMEM((B + L,), jnp.int32),
            pltpu.VMEM((B + L,), jnp.float32),
            pltpu.VMEM((B + L,), jnp.float32),
            pltpu.VMEM((B + L,), jnp.float32),
            pltpu.VMEM((B + L,), jnp.float32),
            pltpu.VMEM((B, FW), jnp.float32),
            pltpu.VMEM((B, FW), jnp.float32),
            pltpu.VMEM((B, FW), jnp.float32),
            pltpu.VMEM((B, FW), jnp.float32),
            pltpu.VMEM((B, FW), jnp.float32),
            pltpu.VMEM((B, FW), jnp.float32),
            pltpu.VMEM((F,), jnp.float32),
            pltpu.VMEM((OWN, FW), jnp.float32),
            pltpu.SemaphoreType.DMA,
            pltpu.SemaphoreType.DMA,
            pltpu.SemaphoreType.DMA,
            pltpu.SemaphoreType.DMA,
            pltpu.SemaphoreType.DMA,
            pltpu.SemaphoreType.DMA,
        ],
    )
    def edge_kernel(asrc, adst, arel, csum_hbm, rec_h, dst_h,
                    out, idh,
                    scan0, scan1, sb, idb,
                    rec0, rec1, srcv0, srcv1, dstv0, dstv1, tyv0, tyv1,
                    lidx0, lidx1, mrow0, mrow1, frow0, frow1,
                    b1a, b1b, b2a, b2b, b3a, b3b, csum_v, acc,
                    sems0, sems1, semf0, semf1, semt0, semt1):
        c = lax.axis_index("c")
        s = lax.axis_index("s")
        wid = c * NS + s
        e_dummy = rec_h.shape[0] - 1
        pltpu.sync_copy(csum_hbm, csum_v)
        iota = lax.iota(jnp.int32, L)
        zero = jnp.zeros((L,), jnp.float32)
        scans = (scan0, scan1)
        semss = (sems0, sems1)
        recs = (rec0, rec1)
        srcvs = (srcv0, srcv1)
        dstvs = (dstv0, dstv1)
        tyvs = (tyv0, tyv1)
        lidxs = (lidx0, lidx1)
        mrows = (mrow0, mrow1)
        frows = (frow0, frow1)
        b1s = (b1a, b1b)
        b2s = (b2a, b2b)
        b3s = (b3a, b3b)
        semfs = (semf0, semf1)
        semts = (semt0, semt1)

        def phase_body(p, ph_carry):
            tile_lo = (p * NT + wid) * OWN
            tp = wid * PH + p

            def zero_body(r, zc):
                for g in range(FW // L):
                    acc[r, pl.ds(g * L, L)] = zero
                return zc

            lax.fori_loop(0, OWN, zero_body, 0)

            def flush_if_full(soff, nf):
                do = soff >= FB

                @pl.when(do)
                def _():
                    pltpu.sync_copy(sb.at[pl.ds(0, FB)],
                                    idh.at[tp, pl.ds(nf * FB, FB)])
                    for j in range(9):
                        sb[pl.ds(j * L, L)] = sb[pl.ds(FB + j * L, L)]

                soff = jnp.where(do, soff - FB, soff)
                nf = jnp.where(do, nf + 1, nf)
                return soff, nf

            def fire_scan(k, ss):
                kc = jnp.minimum(k, n_scan - 1)
                pltpu.async_copy(dst_h.at[pl.ds(kc * SC_CHUNK, SC_CHUNK)],
                                 scans[ss], semss[ss])

            def wait_scan(ss):
                pltpu.make_async_copy(dst_h.at[pl.ds(0, SC_CHUNK)],
                                      scans[ss], semss[ss]).wait()

            def process_scan(ss, k, carry):
                base = k * SC_CHUNK

                def sub(j, car):
                    soff, nf = car
                    for i in range(8):
                        off = j * 128 + i * L
                        dv = scans[ss][pl.ds(off, L)]
                        lv = dv - tile_lo
                        m = (lv >= 0) & (lv < OWN)
                        idv = iota + (base + off)
                        plsc.store_compressed(sb.at[pl.ds(soff, L)], idv,
                                              mask=m)
                        soff = soff + plsc.all_reduce_population_count(m)[0]
                    return flush_if_full(soff, nf)

                return lax.fori_loop(0, SC_CHUNK // 128, sub, carry)

            fire_scan(jnp.int32(0), 0)

            def scan_pair(q, carry):
                k = 2 * q
                wait_scan(0)
                fire_scan(k + 1, 1)
                carry = process_scan(0, k, carry)
                wait_scan(1)
                fire_scan(k + 2, 0)
                carry = process_scan(1, k + 1, carry)
                return carry

            soff, nf = lax.fori_loop(0, n_scan // 2, scan_pair,
                                     (jnp.int32(0), jnp.int32(0)))
            wait_scan(0)
            dummy = jnp.full((L,), e_dummy, jnp.int32)
            for j in range(FB // L):
                sb[pl.ds(soff + j * L, L)] = dummy
            pltpu.sync_copy(sb.at[pl.ds(0, FB)],
                            idh.at[tp, pl.ds(nf * FB, FB)])
            trips = nf + 1
            nchunks = trips * CPB

            def fire_fields(jc, fs):
                sl = idb.at[pl.ds(lax.rem(jc, CPB) * B, B)]
                pltpu.async_copy(rec_h.at[sl], recs[fs], semfs[fs])

            def wait_fields(fs):
                pltpu.make_async_copy(rec_h.at[pl.ds(0, B)], recs[fs],
                                      semfs[fs]).wait()

            def compute_idx(fs):
                for u in range(B // L):
                    rows_ = iota + u * L
                    off = u * L
                    sv = plsc.load_gather(
                        recs[fs], [rows_, jnp.zeros((L,), jnp.int32)])
                    dv = plsc.load_gather(
                        recs[fs], [rows_, jnp.full((L,), 1, jnp.int32)])
                    tv = plsc.load_gather(
                        recs[fs], [rows_, jnp.full((L,), 2, jnp.int32)])
                    fv = plsc.load_gather(
                        recs[fs], [rows_, jnp.full((L,), 3, jnp.int32)])
                    srcvs[fs][pl.ds(off, L)] = sv
                    dstvs[fs][pl.ds(off, L)] = dv
                    tyvs[fs][pl.ds(off, L)] = tv
                    lv = dv - tile_lo
                    m = (lv >= 0) & (lv < OWN)
                    lidxs[fs][pl.ds(off, L)] = jnp.where(m, lv, 0)
                    mrows[fs][pl.ds(off, L)] = jnp.where(m, 1.0, 0.0)
                    frows[fs][pl.ds(off, L)] = plsc.bitcast(fv, jnp.float32)

            def fire_tables(fs):
                pltpu.async_copy(asrc.at[srcvs[fs]], b1s[fs], semts[fs])
                pltpu.async_copy(adst.at[dstvs[fs]], b2s[fs], semts[fs])
                pltpu.async_copy(arel.at[tyvs[fs]], b3s[fs], semts[fs])

            def wait_tables(fs):
                pltpu.make_async_copy(asrc.at[pl.ds(0, B)], b1s[fs],
                                      semts[fs]).wait()
                pltpu.make_async_copy(adst.at[pl.ds(0, B)], b2s[fs],
                                      semts[fs]).wait()
                pltpu.make_async_copy(arel.at[pl.ds(0, B)], b3s[fs],
                                      semts[fs]).wait()

            def rows(fs):
                b1, b2, b3 = b1s[fs], b2s[fs], b3s[fs]

                def row_body(r, rc):
                    rl = lidxs[fs][pl.ds(r, L)][0]
                    fb_ = zero + frows[fs][pl.ds(r, L)][0]
                    mb = zero + mrows[fs][pl.ds(r, L)][0]
                    for g in range(F // L):
                        ca = F + g * L
                        ct = g * L
                        a = (b1[r, pl.ds(ca, L)] + b2[r, pl.ds(ca, L)]
                             + b3[r, pl.ds(ca, L)]
                             + fb_ * csum_v[pl.ds(ct, L)])
                        w = jnp.exp(jnp.maximum(a, 0.01 * a)) * mb
                        t = (b1[r, pl.ds(ct, L)] + b2[r, pl.ds(ct, L)]
                             + b3[r, pl.ds(ct, L)]) * w
                        plsc.addupdate(acc.at[rl, pl.ds(ct, L)], t)
                        plsc.addupdate(acc.at[rl, pl.ds(ca, L)], w)
                    return rc

                lax.fori_loop(0, B, row_body, 0)

            def slot(k, sa):
                sbx = 1 - sa
                wait_fields(sbx)
                compute_idx(sbx)
                fire_tables(sbx)
                wait_tables(sa)

                @pl.when(lax.rem(k + 2, CPB) == 0)
                def _():
                    bi = jnp.minimum((k + 2) // CPB, trips - 1)
                    pltpu.sync_copy(idh.at[tp, pl.ds(bi * FB, FB)], idb)

                jc = jnp.minimum(k + 2, nchunks - 1)
                fire_fields(jc, sa)
                rows(sa)

            pltpu.sync_copy(idh.at[tp, pl.ds(0, FB)], idb)
            fire_fields(jnp.int32(0), 0)
            fire_fields(jnp.minimum(jnp.int32(1), nchunks - 1), 1)
            wait_fields(0)
            compute_idx(0)
            fire_tables(0)

            def pair_cond(k):
                return k < nchunks

            def pair_body(k):
                slot(k, 0)
                slot(k + 1, 1)
                return k + 2

            lax.while_loop(pair_cond, pair_body, jnp.int32(0))
            wait_fields(1)
            wait_tables(0)

            pltpu.sync_copy(acc, out.at[pl.ds(tile_lo, OWN)])
            return ph_carry

        lax.fori_loop(0, PH, phase_body, 0)

    return edge_kernel


def kernel(node, rel, edge_index, edge_type, fre, norm,
           w_triplet, w_quad, loop_weight, evolve_loop_weight):
    n = node.shape[0]
    e = edge_index.shape[1]
    rblk = n // RB

    asrc, adst, arel, csum, lmat, elmat = pl.pallas_call(
        _dense_body,
        grid=(RB,),
        in_specs=[
            pl.BlockSpec((rblk, F), lambda i: (i, 0)),
            pl.BlockSpec(rel.shape, lambda i: (0, 0)),
            pl.BlockSpec((3 * F, F), lambda i: (0, 0)),
            pl.BlockSpec((F, F), lambda i: (0, 0)),
            pl.BlockSpec((F, F), lambda i: (0, 0)),
            pl.BlockSpec((F, F), lambda i: (0, 0)),
        ],
        out_specs=[
            pl.BlockSpec((rblk, FW), lambda i: (i, 0)),
            pl.BlockSpec((rblk, FW), lambda i: (i, 0)),
            pl.BlockSpec((rel.shape[0], FW), lambda i: (0, 0)),
            pl.BlockSpec((1, F), lambda i: (0, 0)),
            pl.BlockSpec((rblk, F), lambda i: (i, 0)),
            pl.BlockSpec((rblk, F), lambda i: (i, 0)),
        ],
        out_shape=[
            jax.ShapeDtypeStruct((n, FW), jnp.float32),
            jax.ShapeDtypeStruct((n, FW), jnp.float32),
            jax.ShapeDtypeStruct((rel.shape[0], FW), jnp.float32),
            jax.ShapeDtypeStruct((1, F), jnp.float32),
            jax.ShapeDtypeStruct((n, F), jnp.float32),
            jax.ShapeDtypeStruct((n, F), jnp.float32),
        ],
    )(node, rel, w_triplet, w_quad, loop_weight, evolve_loop_weight)

    sentinel = jnp.int32(NT * PH * OWN + 7)
    step2 = 2 * SC_CHUNK
    e_pad = ((e + 1 + step2 - 1) // step2) * step2
    src_p = jnp.concatenate([edge_index[0], jnp.zeros((e_pad - e,), jnp.int32)])
    dst_p = jnp.concatenate([edge_index[1],
                             jnp.full((e_pad - e,), sentinel, jnp.int32)])
    ty_p = jnp.concatenate([edge_type, jnp.zeros((e_pad - e,), jnp.int32)])
    fre_p = jnp.concatenate([fre, jnp.zeros((e_pad - e,), jnp.float32)])
    rec_p = jnp.pad(
        jnp.stack(
            [src_p, dst_p, ty_p, lax.bitcast_convert_type(fre_p, jnp.int32)],
            axis=1),
        ((0, 0), (0, 124)))
    cap = (e // FB + 2) * FB

    nd, _ = _make_edge_kernel(e_pad // SC_CHUNK, cap)(
        asrc, adst, arel, csum.reshape(F), rec_p, dst_p)
    nd = nd[:n]

    h = pl.pallas_call(
        _final_body,
        grid=(RB,),
        in_specs=[
            pl.BlockSpec((rblk, FW), lambda i: (i, 0)),
            pl.BlockSpec((rblk, 1), lambda i: (i, 0)),
            pl.BlockSpec((rblk, F), lambda i: (i, 0)),
            pl.BlockSpec((rblk, F), lambda i: (i, 0)),
        ],
        out_specs=pl.BlockSpec((rblk, F), lambda i: (i, 0)),
        out_shape=jax.ShapeDtypeStruct((n, F), jnp.float32),
    )(nd, norm, lmat, elmat)
    return h


# speedup vs baseline: 1.4680x; 1.4680x over previous
import functools

import jax
import jax.numpy as jnp
from jax import lax
from jax.experimental import pallas as pl
from jax.experimental.pallas import tpu as pltpu
from jax.experimental.pallas import tpu_sc as plsc

F = 256
FW = 512
NC, NS, L = 2, 16, 16
NT = NC * NS
PH = 2
OWN = 160
B = 16
FB = 128
SBL = 416
SC_CHUNK = 256
RB = 10


def _dense_body(node_ref, rel_ref, wt_ref, wq_ref, lw_ref, elw_ref,
                asrc_ref, adst_ref, arel_ref, csum_ref, l_ref, el_ref):
    wq = wq_ref[...]
    nb = node_ref[...]
    w1 = wt_ref[0:F, :]
    w2 = wt_ref[F:2 * F, :]
    w3 = wt_ref[2 * F:3 * F, :]
    p1 = jnp.dot(nb, w1, preferred_element_type=jnp.float32)
    asrc_ref[:, 0:F] = p1
    asrc_ref[:, F:FW] = jnp.dot(p1, wq, preferred_element_type=jnp.float32)
    p3 = jnp.dot(nb, w3, preferred_element_type=jnp.float32)
    adst_ref[:, 0:F] = p3
    adst_ref[:, F:FW] = jnp.dot(p3, wq, preferred_element_type=jnp.float32)
    pr = jnp.dot(rel_ref[...], w2, preferred_element_type=jnp.float32)
    arel_ref[:, 0:F] = pr
    arel_ref[:, F:FW] = jnp.dot(pr, wq, preferred_element_type=jnp.float32)
    csum_ref[...] = jnp.sum(wq, axis=0, keepdims=True)
    l_ref[...] = jnp.dot(nb, lw_ref[...], preferred_element_type=jnp.float32)
    el_ref[...] = jnp.dot(nb, elw_ref[...], preferred_element_type=jnp.float32)


def _final_body(nd_ref, norm_ref, l_ref, el_ref, h_ref):
    nd = nd_ref[...]
    num = nd[:, 0:F]
    den = nd[:, F:FW]
    agg = jnp.where(den > 0, num / jnp.maximum(den, 1e-30), 0.0)
    loop = jnp.where(den[:, 0:1] > 0, l_ref[...], el_ref[...])
    h_ref[...] = agg * norm_ref[...] + loop


def _make_edge_kernel(n_scan, cap):
    mesh = plsc.VectorSubcoreMesh(
        core_axis_name="c", subcore_axis_name="s",
        num_cores=NC, num_subcores=NS)

    @functools.partial(
        pl.kernel,
        out_type=(
            jax.ShapeDtypeStruct((NT * PH * OWN, FW), jnp.float32),
            jax.ShapeDtypeStruct((NT * PH, cap), jnp.int32),
        ),
        mesh=mesh,
        compiler_params=pltpu.CompilerParams(needs_layout_passes=False),
        scratch_types=[
            pltpu.VMEM((SC_CHUNK,), jnp.int32),
            pltpu.VMEM((SBL,), jnp.int32),
            pltpu.VMEM((FB,), jnp.int32),
            pltpu.VMEM((B,), jnp.int32),
            pltpu.VMEM((B,), jnp.int32),
            pltpu.VMEM((B,), jnp.int32),
            pltpu.VMEM((B + L,), jnp.int32),
            pltpu.VMEM((B + L,), jnp.float32),
            pltpu.VMEM((B + L,), jnp.float32),
            pltpu.VMEM((B, FW), jnp.float32),
            pltpu.VMEM((B, FW), jnp.float32),
            pltpu.VMEM((B, FW), jnp.float32),
            pltpu.VMEM((F,), jnp.float32),
            pltpu.VMEM((OWN, FW), jnp.float32),
            pltpu.SemaphoreType.DMA,
            pltpu.SemaphoreType.DMA,
            pltpu.SemaphoreType.DMA,
            pltpu.SemaphoreType.DMA,
        ],
    )
    def edge_kernel(asrc, adst, arel, csum_hbm, src_h, dst_h, ty_h, fre_h,
                    out, idh,
                    scan_v, sb, idb, src_v, dst_v, ty_v,
                    lidx_v, fre_v, mrow_v, b1, b2, b3, csum_v, acc,
                    semr, sem1, sem2, sem3):
        c = lax.axis_index("c")
        s = lax.axis_index("s")
        wid = c * NS + s
        e_dummy = src_h.shape[0] - 1
        pltpu.sync_copy(csum_hbm, csum_v)
        iota = lax.iota(jnp.int32, L)
        zero = jnp.zeros((L,), jnp.float32)

        def phase_body(p, ph_carry):
            tile_lo = (p * NT + wid) * OWN
            tp = wid * PH + p

            def zero_body(r, zc):
                for g in range(FW // L):
                    acc[r, pl.ds(g * L, L)] = zero
                return zc

            lax.fori_loop(0, OWN, zero_body, 0)

            def flush_if_full(soff, nf):
                do = soff >= FB

                @pl.when(do)
                def _():
                    pltpu.sync_copy(sb.at[pl.ds(0, FB)],
                                    idh.at[tp, pl.ds(nf * FB, FB)])
                    for j in range(9):
                        sb[pl.ds(j * L, L)] = sb[pl.ds(FB + j * L, L)]

                soff = jnp.where(do, soff - FB, soff)
                nf = jnp.where(do, nf + 1, nf)
                return soff, nf

            def scan_body(k, carry):
                soff, nf = carry
                pltpu.sync_copy(dst_h.at[pl.ds(k * SC_CHUNK, SC_CHUNK)],
                                scan_v)
                for i in range(SC_CHUNK // L):
                    dv = scan_v[pl.ds(i * L, L)]
                    lv = dv - tile_lo
                    m = (lv >= 0) & (lv < OWN)
                    idv = iota + (k * SC_CHUNK + i * L)
                    plsc.store_compressed(sb.at[pl.ds(soff, L)], idv,
                                          mask=m)
                    soff = soff + plsc.all_reduce_population_count(m)[0]
                soff, nf = flush_if_full(soff, nf)
                soff, nf = flush_if_full(soff, nf)
                return soff, nf

            soff, nf = lax.fori_loop(0, n_scan, scan_body,
                                     (jnp.int32(0), jnp.int32(0)))
            dummy = jnp.full((L,), e_dummy, jnp.int32)
            for j in range(FB // L):
                sb[pl.ds(soff + j * L, L)] = dummy
            pltpu.sync_copy(sb.at[pl.ds(0, FB)],
                            idh.at[tp, pl.ds(nf * FB, FB)])
            trips = nf + 1

            def block_cond(bi):
                return bi < trips

            def block_body(bi):
                pltpu.sync_copy(idh.at[tp, pl.ds(bi * FB, FB)], idb)
                for j in range(FB // B):
                    idsl = idb.at[pl.ds(j * B, B)]
                    cp0 = pltpu.async_copy(src_h.at[idsl], src_v, semr)
                    cp1 = pltpu.async_copy(dst_h.at[idsl], dst_v, sem1)
                    cp2 = pltpu.async_copy(ty_h.at[idsl], ty_v, sem2)
                    cp3 = pltpu.async_copy(fre_h.at[idsl],
                                           fre_v.at[pl.ds(0, B)], sem3)
                    cp0.wait()
                    cp1.wait()
                    cp2.wait()
                    cp3.wait()
                    dstv = dst_v[pl.ds(0, L)]
                    lv = dstv - tile_lo
                    m = (lv >= 0) & (lv < OWN)
                    lidx_v[pl.ds(0, L)] = jnp.where(m, lv, 0)
                    mrow_v[pl.ds(0, L)] = jnp.where(m, 1.0, 0.0)
                    cg1 = pltpu.async_copy(asrc.at[src_v], b1, sem1)
                    cg2 = pltpu.async_copy(adst.at[dst_v], b2, sem2)
                    cg3 = pltpu.async_copy(arel.at[ty_v], b3, sem3)
                    cg1.wait()
                    cg2.wait()
                    cg3.wait()

                    def row_body(r, rc):
                        rl = lidx_v[pl.ds(r, L)][0]
                        fb_ = zero + fre_v[pl.ds(r, L)][0]
                        mb = zero + mrow_v[pl.ds(r, L)][0]
                        for g in range(F // L):
                            ca = F + g * L
                            ct = g * L
                            a = (b1[r, pl.ds(ca, L)] + b2[r, pl.ds(ca, L)]
                                 + b3[r, pl.ds(ca, L)]
                                 + fb_ * csum_v[pl.ds(ct, L)])
                            w = jnp.exp(jnp.maximum(a, 0.01 * a)) * mb
                            t = (b1[r, pl.ds(ct, L)] + b2[r, pl.ds(ct, L)]
                                 + b3[r, pl.ds(ct, L)]) * w
                            plsc.addupdate(acc.at[rl, pl.ds(ct, L)], t)
                            plsc.addupdate(acc.at[rl, pl.ds(ca, L)], w)
                        return rc

                    lax.fori_loop(0, B, row_body, 0)
                return bi + 1

            lax.while_loop(block_cond, block_body, jnp.int32(0))

            pltpu.sync_copy(acc, out.at[pl.ds(tile_lo, OWN)])
            return ph_carry

        lax.fori_loop(0, PH, phase_body, 0)

    return edge_kernel


def kernel(node, rel, edge_index, edge_type, fre, norm,
           w_triplet, w_quad, loop_weight, evolve_loop_weight):
    n = node.shape[0]
    e = edge_index.shape[1]
    rblk = n // RB

    asrc, adst, arel, csum, lmat, elmat = pl.pallas_call(
        _dense_body,
        grid=(RB,),
        in_specs=[
            pl.BlockSpec((rblk, F), lambda i: (i, 0)),
            pl.BlockSpec(rel.shape, lambda i: (0, 0)),
            pl.BlockSpec((3 * F, F), lambda i: (0, 0)),
            pl.BlockSpec((F, F), lambda i: (0, 0)),
            pl.BlockSpec((F, F), lambda i: (0, 0)),
            pl.BlockSpec((F, F), lambda i: (0, 0)),
        ],
        out_specs=[
            pl.BlockSpec((rblk, FW), lambda i: (i, 0)),
            pl.BlockSpec((rblk, FW), lambda i: (i, 0)),
            pl.BlockSpec((rel.shape[0], FW), lambda i: (0, 0)),
            pl.BlockSpec((1, F), lambda i: (0, 0)),
            pl.BlockSpec((rblk, F), lambda i: (i, 0)),
            pl.BlockSpec((rblk, F), lambda i: (i, 0)),
        ],
        out_shape=[
            jax.ShapeDtypeStruct((n, FW), jnp.float32),
            jax.ShapeDtypeStruct((n, FW), jnp.float32),
            jax.ShapeDtypeStruct((rel.shape[0], FW), jnp.float32),
            jax.ShapeDtypeStruct((1, F), jnp.float32),
            jax.ShapeDtypeStruct((n, F), jnp.float32),
            jax.ShapeDtypeStruct((n, F), jnp.float32),
        ],
    )(node, rel, w_triplet, w_quad, loop_weight, evolve_loop_weight)

    sentinel = jnp.int32(NT * PH * OWN + 7)
    e_pad = e + SC_CHUNK
    src_p = jnp.concatenate([edge_index[0], jnp.zeros((e_pad - e,), jnp.int32)])
    dst_p = jnp.concatenate([edge_index[1],
                             jnp.full((e_pad - e,), sentinel, jnp.int32)])
    ty_p = jnp.concatenate([edge_type, jnp.zeros((e_pad - e,), jnp.int32)])
    fre_p = jnp.concatenate([fre, jnp.zeros((e_pad - e,), jnp.float32)])
    cap = (e // FB + 2) * FB

    nd, _ = _make_edge_kernel(e_pad // SC_CHUNK, cap)(
        asrc, adst, arel, csum.reshape(F), src_p, dst_p, ty_p, fre_p)
    nd = nd[:n]

    h = pl.pallas_call(
        _final_body,
        grid=(RB,),
        in_specs=[
            pl.BlockSpec((rblk, FW), lambda i: (i, 0)),
            pl.BlockSpec((rblk, 1), lambda i: (i, 0)),
            pl.BlockSpec((rblk, F), lambda i: (i, 0)),
            pl.BlockSpec((rblk, F), lambda i: (i, 0)),
        ],
        out_specs=pl.BlockSpec((rblk, F), lambda i: (i, 0)),
        out_shape=jax.ShapeDtypeStruct((n, F), jnp.float32),
    )(nd, norm, lmat, elmat)
    return h


# speedup vs baseline: 1.4741x; 1.0042x over previous
import functools

import jax
import jax.numpy as jnp
from jax import lax
from jax.experimental import pallas as pl
from jax.experimental.pallas import tpu as pltpu
from jax.experimental.pallas import tpu_sc as plsc

F = 256
FW = 512
NC, NS, L = 2, 16, 16
NT = NC * NS
PH = 2
OWN = 160
B = 24
FB = 384
SBL = 1024
SC_CHUNK = 256
RB = 10


def _dense_body(node_ref, rel_ref, wt_ref, wq_ref, lw_ref, elw_ref,
                asrc_ref, adst_ref, arel_ref, csum_ref, l_ref, el_ref):
    wq = wq_ref[...]
    nb = node_ref[...]
    w1 = wt_ref[0:F, :]
    w2 = wt_ref[F:2 * F, :]
    w3 = wt_ref[2 * F:3 * F, :]
    p1 = jnp.dot(nb, w1, preferred_element_type=jnp.float32)
    asrc_ref[:, 0:F] = p1
    asrc_ref[:, F:FW] = jnp.dot(p1, wq, preferred_element_type=jnp.float32)
    p3 = jnp.dot(nb, w3, preferred_element_type=jnp.float32)
    adst_ref[:, 0:F] = p3
    adst_ref[:, F:FW] = jnp.dot(p3, wq, preferred_element_type=jnp.float32)
    pr = jnp.dot(rel_ref[...], w2, preferred_element_type=jnp.float32)
    arel_ref[:, 0:F] = pr
    arel_ref[:, F:FW] = jnp.dot(pr, wq, preferred_element_type=jnp.float32)
    csum_ref[...] = jnp.sum(wq, axis=0, keepdims=True)
    l_ref[...] = jnp.dot(nb, lw_ref[...], preferred_element_type=jnp.float32)
    el_ref[...] = jnp.dot(nb, elw_ref[...], preferred_element_type=jnp.float32)


def _final_body(nd_ref, norm_ref, l_ref, el_ref, h_ref):
    nd = nd_ref[...]
    num = nd[:, 0:F]
    den = nd[:, F:FW]
    agg = jnp.where(den > 0, num / jnp.maximum(den, 1e-30), 0.0)
    loop = jnp.where(den[:, 0:1] > 0, l_ref[...], el_ref[...])
    h_ref[...] = agg * norm_ref[...] + loop


def _make_edge_kernel(n_scan, cap):
    mesh = plsc.VectorSubcoreMesh(
        core_axis_name="c", subcore_axis_name="s",
        num_cores=NC, num_subcores=NS)

    @functools.partial(
        pl.kernel,
        out_type=(
            jax.ShapeDtypeStruct((NT * PH * OWN, FW), jnp.float32),
            jax.ShapeDtypeStruct((NT * PH, cap), jnp.int32),
        ),
        mesh=mesh,
        compiler_params=pltpu.CompilerParams(needs_layout_passes=False),
        scratch_types=[
            pltpu.VMEM((SC_CHUNK,), jnp.int32),
            pltpu.VMEM((SBL,), jnp.int32),
            pltpu.VMEM((FB,), jnp.int32),
            pltpu.VMEM((B,), jnp.int32),
            pltpu.VMEM((B,), jnp.int32),
            pltpu.VMEM((B,), jnp.int32),
            pltpu.VMEM((B + L,), jnp.int32),
            pltpu.VMEM((B + L,), jnp.float32),
            pltpu.VMEM((B + L,), jnp.float32),
            pltpu.VMEM((B, FW), jnp.float32),
            pltpu.VMEM((B, FW), jnp.float32),
            pltpu.VMEM((B, FW), jnp.float32),
            pltpu.VMEM((F,), jnp.float32),
            pltpu.VMEM((OWN, FW), jnp.float32),
            pltpu.SemaphoreType.DMA,
            pltpu.SemaphoreType.DMA,
            pltpu.SemaphoreType.DMA,
            pltpu.SemaphoreType.DMA,
        ],
    )
    def edge_kernel(asrc, adst, arel, csum_hbm, src_h, dst_h, ty_h, fre_h,
                    out, idh,
                    scan_v, sb, idb, src_v, dst_v, ty_v,
                    lidx_v, fre_v, mrow_v, b1, b2, b3, csum_v, acc,
                    semr, sem1, sem2, sem3):
        c = lax.axis_index("c")
        s = lax.axis_index("s")
        wid = c * NS + s
        e_dummy = src_h.shape[0] - 1
        pltpu.sync_copy(csum_hbm, csum_v)
        iota = lax.iota(jnp.int32, L)
        zero = jnp.zeros((L,), jnp.float32)

        def phase_body(p, ph_carry):
            tile_lo = (p * NT + wid) * OWN
            tp = wid * PH + p

            def zero_body(r, zc):
                for g in range(FW // L):
                    acc[r, pl.ds(g * L, L)] = zero
                return zc

            lax.fori_loop(0, OWN, zero_body, 0)

            def flush_if_full(soff, nf):
                do = soff >= FB

                @pl.when(do)
                def _():
                    pltpu.sync_copy(sb.at[pl.ds(0, FB)],
                                    idh.at[tp, pl.ds(pl.multiple_of(nf * FB, 8), FB)])
                    for j in range(16):
                        sb[pl.ds(j * L, L)] = sb[pl.ds(FB + j * L, L)]

                soff = jnp.where(do, soff - FB, soff)
                nf = jnp.where(do, nf + 1, nf)
                return soff, nf

            def scan_body(k, carry):
                soff, nf = carry
                pltpu.sync_copy(dst_h.at[pl.ds(k * SC_CHUNK, SC_CHUNK)],
                                scan_v)
                for i in range(SC_CHUNK // L):
                    dv = scan_v[pl.ds(i * L, L)]
                    lv = dv - tile_lo
                    m = (lv >= 0) & (lv < OWN)
                    idv = iota + (k * SC_CHUNK + i * L)
                    plsc.store_compressed(sb.at[pl.ds(soff, L)], idv,
                                          mask=m)
                    soff = soff + plsc.all_reduce_population_count(m)[0]
                soff, nf = flush_if_full(soff, nf)
                return soff, nf

            soff, nf = lax.fori_loop(0, n_scan, scan_body,
                                     (jnp.int32(0), jnp.int32(0)))
            dummy = jnp.full((L,), e_dummy, jnp.int32)
            for j in range(FB // L):
                sb[pl.ds(soff + j * L, L)] = dummy
            pltpu.sync_copy(sb.at[pl.ds(0, FB)],
                            idh.at[tp, pl.ds(pl.multiple_of(nf * FB, 8), FB)])
            trips = nf + 1

            def block_cond(bi):
                return bi < trips

            def block_body(bi):
                pltpu.sync_copy(
                    idh.at[tp, pl.ds(pl.multiple_of(bi * FB, 8), FB)], idb)
                for j in range(FB // B):
                    idsl = idb.at[pl.ds(j * B, B)]
                    cp0 = pltpu.async_copy(src_h.at[idsl], src_v, semr)
                    cp1 = pltpu.async_copy(dst_h.at[idsl], dst_v, sem1)
                    cp2 = pltpu.async_copy(ty_h.at[idsl], ty_v, sem2)
                    cp3 = pltpu.async_copy(fre_h.at[idsl],
                                           fre_v.at[pl.ds(0, B)], sem3)
                    cp0.wait()
                    cp1.wait()
                    cp2.wait()
                    cp3.wait()
                    for u in (0, B - L):
                        dstv = dst_v[pl.ds(u, L)]
                        lv = dstv - tile_lo
                        m = (lv >= 0) & (lv < OWN)
                        lidx_v[pl.ds(u, L)] = jnp.where(m, lv, 0)
                        mrow_v[pl.ds(u, L)] = jnp.where(m, 1.0, 0.0)
                    cg1 = pltpu.async_copy(asrc.at[src_v], b1, sem1)
                    cg2 = pltpu.async_copy(adst.at[dst_v], b2, sem2)
                    cg3 = pltpu.async_copy(arel.at[ty_v], b3, sem3)
                    cg1.wait()
                    cg2.wait()
                    cg3.wait()

                    def row_body(r, rc):
                        rl = lidx_v[pl.ds(r, L)][0]
                        fb_ = zero + fre_v[pl.ds(r, L)][0]
                        mb = zero + mrow_v[pl.ds(r, L)][0]
                        for g in range(F // L):
                            ca = F + g * L
                            ct = g * L
                            a = (b1[r, pl.ds(ca, L)] + b2[r, pl.ds(ca, L)]
                                 + b3[r, pl.ds(ca, L)]
                                 + fb_ * csum_v[pl.ds(ct, L)])
                            w = jnp.exp(jnp.maximum(a, 0.01 * a)) * mb
                            t = (b1[r, pl.ds(ct, L)] + b2[r, pl.ds(ct, L)]
                                 + b3[r, pl.ds(ct, L)]) * w
                            plsc.addupdate(acc.at[rl, pl.ds(ct, L)], t)
                            plsc.addupdate(acc.at[rl, pl.ds(ca, L)], w)
                        return rc

                    lax.fori_loop(0, B, row_body, 0)
                return bi + 1

            lax.while_loop(block_cond, block_body, jnp.int32(0))

            pltpu.sync_copy(acc, out.at[pl.ds(tile_lo, OWN)])
            return ph_carry

        lax.fori_loop(0, PH, phase_body, 0)

    return edge_kernel


def kernel(node, rel, edge_index, edge_type, fre, norm,
           w_triplet, w_quad, loop_weight, evolve_loop_weight):
    n = node.shape[0]
    e = edge_index.shape[1]
    rblk = n // RB

    asrc, adst, arel, csum, lmat, elmat = pl.pallas_call(
        _dense_body,
        grid=(RB,),
        in_specs=[
            pl.BlockSpec((rblk, F), lambda i: (i, 0)),
            pl.BlockSpec(rel.shape, lambda i: (0, 0)),
            pl.BlockSpec((3 * F, F), lambda i: (0, 0)),
            pl.BlockSpec((F, F), lambda i: (0, 0)),
            pl.BlockSpec((F, F), lambda i: (0, 0)),
            pl.BlockSpec((F, F), lambda i: (0, 0)),
        ],
        out_specs=[
            pl.BlockSpec((rblk, FW), lambda i: (i, 0)),
            pl.BlockSpec((rblk, FW), lambda i: (i, 0)),
            pl.BlockSpec((rel.shape[0], FW), lambda i: (0, 0)),
            pl.BlockSpec((1, F), lambda i: (0, 0)),
            pl.BlockSpec((rblk, F), lambda i: (i, 0)),
            pl.BlockSpec((rblk, F), lambda i: (i, 0)),
        ],
        out_shape=[
            jax.ShapeDtypeStruct((n, FW), jnp.float32),
            jax.ShapeDtypeStruct((n, FW), jnp.float32),
            jax.ShapeDtypeStruct((rel.shape[0], FW), jnp.float32),
            jax.ShapeDtypeStruct((1, F), jnp.float32),
            jax.ShapeDtypeStruct((n, F), jnp.float32),
            jax.ShapeDtypeStruct((n, F), jnp.float32),
        ],
    )(node, rel, w_triplet, w_quad, loop_weight, evolve_loop_weight)

    sentinel = jnp.int32(NT * PH * OWN + 7)
    e_pad = e + SC_CHUNK
    src_p = jnp.concatenate([edge_index[0], jnp.zeros((e_pad - e,), jnp.int32)])
    dst_p = jnp.concatenate([edge_index[1],
                             jnp.full((e_pad - e,), sentinel, jnp.int32)])
    ty_p = jnp.concatenate([edge_type, jnp.zeros((e_pad - e,), jnp.int32)])
    fre_p = jnp.concatenate([fre, jnp.zeros((e_pad - e,), jnp.float32)])
    cap = -(-((e // FB + 2) * FB) // 128) * 128

    nd, _ = _make_edge_kernel(e_pad // SC_CHUNK, cap)(
        asrc, adst, arel, csum.reshape(F), src_p, dst_p, ty_p, fre_p)
    nd = nd[:n]

    h = pl.pallas_call(
        _final_body,
        grid=(RB,),
        in_specs=[
            pl.BlockSpec((rblk, FW), lambda i: (i, 0)),
            pl.BlockSpec((rblk, 1), lambda i: (i, 0)),
            pl.BlockSpec((rblk, F), lambda i: (i, 0)),
            pl.BlockSpec((rblk, F), lambda i: (i, 0)),
        ],
        out_specs=pl.BlockSpec((rblk, F), lambda i: (i, 0)),
        out_shape=jax.ShapeDtypeStruct((n, F), jnp.float32),
    )(nd, norm, lmat, elmat)
    return h


# speedup vs baseline: 1.6154x; 1.0958x over previous
import functools

import jax
import jax.numpy as jnp
from jax import lax
from jax.experimental import pallas as pl
from jax.experimental.pallas import tpu as pltpu
from jax.experimental.pallas import tpu_sc as plsc

F = 256
FW = 512
NC, NS, L = 2, 16, 16
NT = NC * NS
PH = 2
OWN = 160
B = 24
FB = 384
SBL = 1024
SC_CHUNK = 512
RB = 10


def _dense_body(node_ref, rel_ref, wt_ref, wq_ref, lw_ref, elw_ref,
                asrc_ref, adst_ref, arel_ref, csum_ref, l_ref, el_ref):
    wq = wq_ref[...]
    nb = node_ref[...]
    w1 = wt_ref[0:F, :]
    w2 = wt_ref[F:2 * F, :]
    w3 = wt_ref[2 * F:3 * F, :]
    p1 = jnp.dot(nb, w1, preferred_element_type=jnp.float32)
    asrc_ref[:, 0:F] = p1
    asrc_ref[:, F:FW] = jnp.dot(p1, wq, preferred_element_type=jnp.float32)
    p3 = jnp.dot(nb, w3, preferred_element_type=jnp.float32)
    adst_ref[:, 0:F] = p3
    adst_ref[:, F:FW] = jnp.dot(p3, wq, preferred_element_type=jnp.float32)
    pr = jnp.dot(rel_ref[...], w2, preferred_element_type=jnp.float32)
    arel_ref[:, 0:F] = pr
    arel_ref[:, F:FW] = jnp.dot(pr, wq, preferred_element_type=jnp.float32)
    csum_ref[...] = jnp.sum(wq, axis=0, keepdims=True)
    l_ref[...] = jnp.dot(nb, lw_ref[...], preferred_element_type=jnp.float32)
    el_ref[...] = jnp.dot(nb, elw_ref[...], preferred_element_type=jnp.float32)


def _final_body(nd_ref, norm_ref, l_ref, el_ref, h_ref):
    nd = nd_ref[...]
    num = nd[:, 0:F]
    den = nd[:, F:FW]
    agg = jnp.where(den > 0, num / jnp.maximum(den, 1e-30), 0.0)
    loop = jnp.where(den[:, 0:1] > 0, l_ref[...], el_ref[...])
    h_ref[...] = agg * norm_ref[...] + loop


def _make_edge_kernel(n_scan, cap):
    mesh = plsc.VectorSubcoreMesh(
        core_axis_name="c", subcore_axis_name="s",
        num_cores=NC, num_subcores=NS)

    @functools.partial(
        pl.kernel,
        out_type=(
            jax.ShapeDtypeStruct((NT * PH * OWN, FW), jnp.float32),
            jax.ShapeDtypeStruct((NT * PH, cap), jnp.int32),
        ),
        mesh=mesh,
        compiler_params=pltpu.CompilerParams(needs_layout_passes=False),
        scratch_types=[
            pltpu.VMEM((SC_CHUNK,), jnp.int32),
            pltpu.VMEM((SBL,), jnp.int32),
            pltpu.VMEM((FB,), jnp.int32),
            pltpu.VMEM((B,), jnp.int32),
            pltpu.VMEM((B,), jnp.int32),
            pltpu.VMEM((B,), jnp.int32),
            pltpu.VMEM((B + L,), jnp.int32),
            pltpu.VMEM((B + L,), jnp.float32),
            pltpu.VMEM((B + L,), jnp.float32),
            pltpu.VMEM((B, FW), jnp.float32),
            pltpu.VMEM((B, FW), jnp.float32),
            pltpu.VMEM((B, FW), jnp.float32),
            pltpu.VMEM((F,), jnp.float32),
            pltpu.VMEM((OWN, FW), jnp.float32),
            pltpu.SemaphoreType.DMA,
            pltpu.SemaphoreType.DMA,
            pltpu.SemaphoreType.DMA,
            pltpu.SemaphoreType.DMA,
        ],
    )
    def edge_kernel(asrc, adst, arel, csum_hbm, src_h, dst_h, ty_h, fre_h,
                    out, idh,
                    scan_v, sb, idb, src_v, dst_v, ty_v,
                    lidx_v, fre_v, mrow_v, b1, b2, b3, csum_v, acc,
                    semr, sem1, sem2, sem3):
        c = lax.axis_index("c")
        s = lax.axis_index("s")
        wid = c * NS + s
        e_dummy = src_h.shape[0] - 1
        pltpu.sync_copy(csum_hbm, csum_v)
        iota = lax.iota(jnp.int32, L)
        zero = jnp.zeros((L,), jnp.float32)

        def phase_body(p, ph_carry):
            tile_lo = (p * NT + wid) * OWN
            tp = wid * PH + p

            def zero_body(r, zc):
                for g in range(FW // L):
                    acc[r, pl.ds(g * L, L)] = zero
                return zc

            lax.fori_loop(0, OWN, zero_body, 0)

            def flush_if_full(soff, nf):
                do = soff >= FB

                @pl.when(do)
                def _():
                    pltpu.sync_copy(sb.at[pl.ds(0, FB)],
                                    idh.at[tp, pl.ds(pl.multiple_of(nf * FB, 8), FB)])
                    for j in range(32):
                        sb[pl.ds(j * L, L)] = sb[pl.ds(FB + j * L, L)]

                soff = jnp.where(do, soff - FB, soff)
                nf = jnp.where(do, nf + 1, nf)
                return soff, nf

            def scan_body(k, carry):
                soff, nf = carry
                pltpu.sync_copy(dst_h.at[pl.ds(k * SC_CHUNK, SC_CHUNK)],
                                scan_v)
                for i in range(SC_CHUNK // L):
                    dv = scan_v[pl.ds(i * L, L)]
                    lv = dv - tile_lo
                    m = (lv >= 0) & (lv < OWN)
                    idv = iota + (k * SC_CHUNK + i * L)
                    plsc.store_compressed(sb.at[pl.ds(soff, L)], idv,
                                          mask=m)
                    soff = soff + plsc.all_reduce_population_count(m)[0]
                soff, nf = flush_if_full(soff, nf)
                soff, nf = flush_if_full(soff, nf)
                return soff, nf

            soff, nf = lax.fori_loop(0, n_scan, scan_body,
                                     (jnp.int32(0), jnp.int32(0)))
            dummy = jnp.full((L,), e_dummy, jnp.int32)
            for j in range(FB // L):
                sb[pl.ds(soff + j * L, L)] = dummy
            pltpu.sync_copy(sb.at[pl.ds(0, FB)],
                            idh.at[tp, pl.ds(pl.multiple_of(nf * FB, 8), FB)])
            trips = nf + 1

            def block_cond(bi):
                return bi < trips

            def block_body(bi):
                pltpu.sync_copy(
                    idh.at[tp, pl.ds(pl.multiple_of(bi * FB, 8), FB)], idb)
                for j in range(FB // B):
                    idsl = idb.at[pl.ds(j * B, B)]
                    cp0 = pltpu.async_copy(src_h.at[idsl], src_v, semr)
                    cp1 = pltpu.async_copy(dst_h.at[idsl], dst_v, sem1)
                    cp2 = pltpu.async_copy(ty_h.at[idsl], ty_v, sem2)
                    cp3 = pltpu.async_copy(fre_h.at[idsl],
                                           fre_v.at[pl.ds(0, B)], sem3)
                    cp0.wait()
                    cp1.wait()
                    cp2.wait()
                    cp3.wait()
                    for u in (0, B - L):
                        dstv = dst_v[pl.ds(u, L)]
                        lv = dstv - tile_lo
                        m = (lv >= 0) & (lv < OWN)
                        lidx_v[pl.ds(u, L)] = jnp.where(m, lv, 0)
                        mrow_v[pl.ds(u, L)] = jnp.where(m, 1.0, 0.0)
                    cg1 = pltpu.async_copy(asrc.at[src_v], b1, sem1)
                    cg2 = pltpu.async_copy(adst.at[dst_v], b2, sem2)
                    cg3 = pltpu.async_copy(arel.at[ty_v], b3, sem3)
                    cg1.wait()
                    cg2.wait()
                    cg3.wait()

                    def row_body(r, rc):
                        rl = lidx_v[pl.ds(r, L)][0]
                        fb_ = zero + fre_v[pl.ds(r, L)][0]
                        mb = zero + mrow_v[pl.ds(r, L)][0]
                        for g in range(F // L):
                            ca = F + g * L
                            ct = g * L
                            a = (b1[r, pl.ds(ca, L)] + b2[r, pl.ds(ca, L)]
                                 + b3[r, pl.ds(ca, L)]
                                 + fb_ * csum_v[pl.ds(ct, L)])
                            w = jnp.exp(jnp.maximum(a, 0.01 * a)) * mb
                            t = (b1[r, pl.ds(ct, L)] + b2[r, pl.ds(ct, L)]
                                 + b3[r, pl.ds(ct, L)]) * w
                            plsc.addupdate(acc.at[rl, pl.ds(ct, L)], t)
                            plsc.addupdate(acc.at[rl, pl.ds(ca, L)], w)
                        return rc

                    lax.fori_loop(0, B, row_body, 0)
                return bi + 1

            lax.while_loop(block_cond, block_body, jnp.int32(0))

            pltpu.sync_copy(acc, out.at[pl.ds(tile_lo, OWN)])
            return ph_carry

        lax.fori_loop(0, PH, phase_body, 0)

    return edge_kernel


def kernel(node, rel, edge_index, edge_type, fre, norm,
           w_triplet, w_quad, loop_weight, evolve_loop_weight):
    n = node.shape[0]
    e = edge_index.shape[1]
    rblk = n // RB

    asrc, adst, arel, csum, lmat, elmat = pl.pallas_call(
        _dense_body,
        grid=(RB,),
        in_specs=[
            pl.BlockSpec((rblk, F), lambda i: (i, 0)),
            pl.BlockSpec(rel.shape, lambda i: (0, 0)),
            pl.BlockSpec((3 * F, F), lambda i: (0, 0)),
            pl.BlockSpec((F, F), lambda i: (0, 0)),
            pl.BlockSpec((F, F), lambda i: (0, 0)),
            pl.BlockSpec((F, F), lambda i: (0, 0)),
        ],
        out_specs=[
            pl.BlockSpec((rblk, FW), lambda i: (i, 0)),
            pl.BlockSpec((rblk, FW), lambda i: (i, 0)),
            pl.BlockSpec((rel.shape[0], FW), lambda i: (0, 0)),
            pl.BlockSpec((1, F), lambda i: (0, 0)),
            pl.BlockSpec((rblk, F), lambda i: (i, 0)),
            pl.BlockSpec((rblk, F), lambda i: (i, 0)),
        ],
        out_shape=[
            jax.ShapeDtypeStruct((n, FW), jnp.float32),
            jax.ShapeDtypeStruct((n, FW), jnp.float32),
            jax.ShapeDtypeStruct((rel.shape[0], FW), jnp.float32),
            jax.ShapeDtypeStruct((1, F), jnp.float32),
            jax.ShapeDtypeStruct((n, F), jnp.float32),
            jax.ShapeDtypeStruct((n, F), jnp.float32),
        ],
    )(node, rel, w_triplet, w_quad, loop_weight, evolve_loop_weight)

    sentinel = jnp.int32(NT * PH * OWN + 7)
    e_pad = -(-(e + 1) // SC_CHUNK) * SC_CHUNK
    src_p = jnp.concatenate([edge_index[0], jnp.zeros((e_pad - e,), jnp.int32)])
    dst_p = jnp.concatenate([edge_index[1],
                             jnp.full((e_pad - e,), sentinel, jnp.int32)])
    ty_p = jnp.concatenate([edge_type, jnp.zeros((e_pad - e,), jnp.int32)])
    fre_p = jnp.concatenate([fre, jnp.zeros((e_pad - e,), jnp.float32)])
    cap = -(-((e // FB + 2) * FB) // 128) * 128

    nd, _ = _make_edge_kernel(e_pad // SC_CHUNK, cap)(
        asrc, adst, arel, csum.reshape(F), src_p, dst_p, ty_p, fre_p)
    nd = nd[:n]

    h = pl.pallas_call(
        _final_body,
        grid=(RB,),
        in_specs=[
            pl.BlockSpec((rblk, FW), lambda i: (i, 0)),
            pl.BlockSpec((rblk, 1), lambda i: (i, 0)),
            pl.BlockSpec((rblk, F), lambda i: (i, 0)),
            pl.BlockSpec((rblk, F), lambda i: (i, 0)),
        ],
        out_specs=pl.BlockSpec((rblk, F), lambda i: (i, 0)),
        out_shape=jax.ShapeDtypeStruct((n, F), jnp.float32),
    )(nd, norm, lmat, elmat)
    return h


# speedup vs baseline: 1.6974x; 1.0508x over previous
import functools

import jax
import jax.numpy as jnp
from jax import lax
from jax.experimental import pallas as pl
from jax.experimental.pallas import tpu as pltpu
from jax.experimental.pallas import tpu_sc as plsc

F = 256
FW = 512
NC, NS, L = 2, 16, 16
NT = NC * NS
PH = 2
OWN = 160
B = 24
FB = 384
SBL = 1536
SC_CHUNK = 1024
RB = 10


def _dense_body(node_ref, rel_ref, wt_ref, wq_ref, lw_ref, elw_ref,
                asrc_ref, adst_ref, arel_ref, csum_ref, l_ref, el_ref):
    wq = wq_ref[...]
    nb = node_ref[...]
    w1 = wt_ref[0:F, :]
    w2 = wt_ref[F:2 * F, :]
    w3 = wt_ref[2 * F:3 * F, :]
    p1 = jnp.dot(nb, w1, preferred_element_type=jnp.float32)
    asrc_ref[:, 0:F] = p1
    asrc_ref[:, F:FW] = jnp.dot(p1, wq, preferred_element_type=jnp.float32)
    p3 = jnp.dot(nb, w3, preferred_element_type=jnp.float32)
    adst_ref[:, 0:F] = p3
    adst_ref[:, F:FW] = jnp.dot(p3, wq, preferred_element_type=jnp.float32)
    pr = jnp.dot(rel_ref[...], w2, preferred_element_type=jnp.float32)
    arel_ref[:, 0:F] = pr
    arel_ref[:, F:FW] = jnp.dot(pr, wq, preferred_element_type=jnp.float32)
    csum_ref[...] = jnp.sum(wq, axis=0, keepdims=True)
    l_ref[...] = jnp.dot(nb, lw_ref[...], preferred_element_type=jnp.float32)
    el_ref[...] = jnp.dot(nb, elw_ref[...], preferred_element_type=jnp.float32)


def _final_body(nd_ref, norm_ref, l_ref, el_ref, h_ref):
    nd = nd_ref[...]
    num = nd[:, 0:F]
    den = nd[:, F:FW]
    agg = jnp.where(den > 0, num / jnp.maximum(den, 1e-30), 0.0)
    loop = jnp.where(den[:, 0:1] > 0, l_ref[...], el_ref[...])
    h_ref[...] = agg * norm_ref[...] + loop


def _make_edge_kernel(n_scan, cap):
    mesh = plsc.VectorSubcoreMesh(
        core_axis_name="c", subcore_axis_name="s",
        num_cores=NC, num_subcores=NS)

    @functools.partial(
        pl.kernel,
        out_type=(
            jax.ShapeDtypeStruct((NT * PH * OWN, FW), jnp.float32),
            jax.ShapeDtypeStruct((NT * PH, cap), jnp.int32),
        ),
        mesh=mesh,
        compiler_params=pltpu.CompilerParams(needs_layout_passes=False),
        scratch_types=[
            pltpu.VMEM((SC_CHUNK,), jnp.int32),
            pltpu.VMEM((SBL,), jnp.int32),
            pltpu.VMEM((FB,), jnp.int32),
            pltpu.VMEM((B,), jnp.int32),
            pltpu.VMEM((B,), jnp.int32),
            pltpu.VMEM((B,), jnp.int32),
            pltpu.VMEM((B + L,), jnp.int32),
            pltpu.VMEM((B + L,), jnp.float32),
            pltpu.VMEM((B + L,), jnp.float32),
            pltpu.VMEM((B, FW), jnp.float32),
            pltpu.VMEM((B, FW), jnp.float32),
            pltpu.VMEM((B, FW), jnp.float32),
            pltpu.VMEM((F,), jnp.float32),
            pltpu.VMEM((OWN, FW), jnp.float32),
            pltpu.SemaphoreType.DMA,
            pltpu.SemaphoreType.DMA,
            pltpu.SemaphoreType.DMA,
            pltpu.SemaphoreType.DMA,
        ],
    )
    def edge_kernel(asrc, adst, arel, csum_hbm, src_h, dst_h, ty_h, fre_h,
                    out, idh,
                    scan_v, sb, idb, src_v, dst_v, ty_v,
                    lidx_v, fre_v, mrow_v, b1, b2, b3, csum_v, acc,
                    semr, sem1, sem2, sem3):
        c = lax.axis_index("c")
        s = lax.axis_index("s")
        wid = c * NS + s
        e_dummy = src_h.shape[0] - 1
        pltpu.sync_copy(csum_hbm, csum_v)
        iota = lax.iota(jnp.int32, L)
        zero = jnp.zeros((L,), jnp.float32)

        def phase_body(p, ph_carry):
            tile_lo = (p * NT + wid) * OWN
            tp = wid * PH + p

            def zero_body(r, zc):
                for g in range(FW // L):
                    acc[r, pl.ds(g * L, L)] = zero
                return zc

            lax.fori_loop(0, OWN, zero_body, 0)

            def flush_if_full(soff, nf):
                do = soff >= FB

                @pl.when(do)
                def _():
                    pltpu.sync_copy(sb.at[pl.ds(0, FB)],
                                    idh.at[tp, pl.ds(pl.multiple_of(nf * FB, 8), FB)])
                    for j in range(64):
                        sb[pl.ds(j * L, L)] = sb[pl.ds(FB + j * L, L)]

                soff = jnp.where(do, soff - FB, soff)
                nf = jnp.where(do, nf + 1, nf)
                return soff, nf

            def scan_body(k, carry):
                soff, nf = carry
                pltpu.sync_copy(dst_h.at[pl.ds(k * SC_CHUNK, SC_CHUNK)],
                                scan_v)
                for i in range(SC_CHUNK // L):
                    dv = scan_v[pl.ds(i * L, L)]
                    lv = dv - tile_lo
                    m = (lv >= 0) & (lv < OWN)
                    idv = iota + (k * SC_CHUNK + i * L)
                    plsc.store_compressed(sb.at[pl.ds(soff, L)], idv,
                                          mask=m)
                    soff = soff + plsc.all_reduce_population_count(m)[0]
                soff, nf = flush_if_full(soff, nf)
                soff, nf = flush_if_full(soff, nf)
                soff, nf = flush_if_full(soff, nf)
                return soff, nf

            soff, nf = lax.fori_loop(0, n_scan, scan_body,
                                     (jnp.int32(0), jnp.int32(0)))
            dummy = jnp.full((L,), e_dummy, jnp.int32)
            for j in range(FB // L):
                sb[pl.ds(soff + j * L, L)] = dummy
            pltpu.sync_copy(sb.at[pl.ds(0, FB)],
                            idh.at[tp, pl.ds(pl.multiple_of(nf * FB, 8), FB)])
            trips = nf + 1

            def block_cond(bi):
                return bi < trips

            def block_body(bi):
                pltpu.sync_copy(
                    idh.at[tp, pl.ds(pl.multiple_of(bi * FB, 8), FB)], idb)
                for j in range(FB // B):
                    idsl = idb.at[pl.ds(j * B, B)]
                    cp0 = pltpu.async_copy(src_h.at[idsl], src_v, semr)
                    cp1 = pltpu.async_copy(dst_h.at[idsl], dst_v, sem1)
                    cp2 = pltpu.async_copy(ty_h.at[idsl], ty_v, sem2)
                    cp3 = pltpu.async_copy(fre_h.at[idsl],
                                           fre_v.at[pl.ds(0, B)], sem3)
                    cp0.wait()
                    cp1.wait()
                    cp2.wait()
                    cp3.wait()
                    for u in (0, B - L):
                        dstv = dst_v[pl.ds(u, L)]
                        lv = dstv - tile_lo
                        m = (lv >= 0) & (lv < OWN)
                        lidx_v[pl.ds(u, L)] = jnp.where(m, lv, 0)
                        mrow_v[pl.ds(u, L)] = jnp.where(m, 1.0, 0.0)
                    cg1 = pltpu.async_copy(asrc.at[src_v], b1, sem1)
                    cg2 = pltpu.async_copy(adst.at[dst_v], b2, sem2)
                    cg3 = pltpu.async_copy(arel.at[ty_v], b3, sem3)
                    cg1.wait()
                    cg2.wait()
                    cg3.wait()

                    def row_body(r, rc):
                        rl = lidx_v[pl.ds(r, L)][0]
                        fb_ = zero + fre_v[pl.ds(r, L)][0]
                        mb = zero + mrow_v[pl.ds(r, L)][0]
                        for g in range(F // L):
                            ca = F + g * L
                            ct = g * L
                            a = (b1[r, pl.ds(ca, L)] + b2[r, pl.ds(ca, L)]
                                 + b3[r, pl.ds(ca, L)]
                                 + fb_ * csum_v[pl.ds(ct, L)])
                            w = jnp.exp(jnp.maximum(a, 0.01 * a)) * mb
                            t = (b1[r, pl.ds(ct, L)] + b2[r, pl.ds(ct, L)]
                                 + b3[r, pl.ds(ct, L)]) * w
                            plsc.addupdate(acc.at[rl, pl.ds(ct, L)], t)
                            plsc.addupdate(acc.at[rl, pl.ds(ca, L)], w)
                        return rc

                    lax.fori_loop(0, B, row_body, 0)
                return bi + 1

            lax.while_loop(block_cond, block_body, jnp.int32(0))

            pltpu.sync_copy(acc, out.at[pl.ds(tile_lo, OWN)])
            return ph_carry

        lax.fori_loop(0, PH, phase_body, 0)

    return edge_kernel


def kernel(node, rel, edge_index, edge_type, fre, norm,
           w_triplet, w_quad, loop_weight, evolve_loop_weight):
    n = node.shape[0]
    e = edge_index.shape[1]
    rblk = n // RB

    asrc, adst, arel, csum, lmat, elmat = pl.pallas_call(
        _dense_body,
        grid=(RB,),
        in_specs=[
            pl.BlockSpec((rblk, F), lambda i: (i, 0)),
            pl.BlockSpec(rel.shape, lambda i: (0, 0)),
            pl.BlockSpec((3 * F, F), lambda i: (0, 0)),
            pl.BlockSpec((F, F), lambda i: (0, 0)),
            pl.BlockSpec((F, F), lambda i: (0, 0)),
            pl.BlockSpec((F, F), lambda i: (0, 0)),
        ],
        out_specs=[
            pl.BlockSpec((rblk, FW), lambda i: (i, 0)),
            pl.BlockSpec((rblk, FW), lambda i: (i, 0)),
            pl.BlockSpec((rel.shape[0], FW), lambda i: (0, 0)),
            pl.BlockSpec((1, F), lambda i: (0, 0)),
            pl.BlockSpec((rblk, F), lambda i: (i, 0)),
            pl.BlockSpec((rblk, F), lambda i: (i, 0)),
        ],
        out_shape=[
            jax.ShapeDtypeStruct((n, FW), jnp.float32),
            jax.ShapeDtypeStruct((n, FW), jnp.float32),
            jax.ShapeDtypeStruct((rel.shape[0], FW), jnp.float32),
            jax.ShapeDtypeStruct((1, F), jnp.float32),
            jax.ShapeDtypeStruct((n, F), jnp.float32),
            jax.ShapeDtypeStruct((n, F), jnp.float32),
        ],
    )(node, rel, w_triplet, w_quad, loop_weight, evolve_loop_weight)

    sentinel = jnp.int32(NT * PH * OWN + 7)
    e_pad = -(-(e + 1) // SC_CHUNK) * SC_CHUNK
    src_p = jnp.concatenate([edge_index[0], jnp.zeros((e_pad - e,), jnp.int32)])
    dst_p = jnp.concatenate([edge_index[1],
                             jnp.full((e_pad - e,), sentinel, jnp.int32)])
    ty_p = jnp.concatenate([edge_type, jnp.zeros((e_pad - e,), jnp.int32)])
    fre_p = jnp.concatenate([fre, jnp.zeros((e_pad - e,), jnp.float32)])
    cap = -(-((e // FB + 2) * FB) // 128) * 128

    nd, _ = _make_edge_kernel(e_pad // SC_CHUNK, cap)(
        asrc, adst, arel, csum.reshape(F), src_p, dst_p, ty_p, fre_p)
    nd = nd[:n]

    h = pl.pallas_call(
        _final_body,
        grid=(RB,),
        in_specs=[
            pl.BlockSpec((rblk, FW), lambda i: (i, 0)),
            pl.BlockSpec((rblk, 1), lambda i: (i, 0)),
            pl.BlockSpec((rblk, F), lambda i: (i, 0)),
            pl.BlockSpec((rblk, F), lambda i: (i, 0)),
        ],
        out_specs=pl.BlockSpec((rblk, F), lambda i: (i, 0)),
        out_shape=jax.ShapeDtypeStruct((n, F), jnp.float32),
    )(nd, norm, lmat, elmat)
    return h
